# baseline (speedup 1.0000x reference)
import jax
import jax.numpy as jnp
from jax import lax
from jax.experimental import pallas as pl
from jax.experimental.pallas import tpu as pltpu
from jax.experimental.pallas import tpu_sc as plsc

_NC = 2
_NS = 16
_NW = _NC * _NS
_L = 16

_C2 = 12800
_PAD_V = 1024000
_TPT = _PAD_V // _NS


def _sigmoid_chunk(vb):
    def body(i, c):
        b = i * (4 * _L)
        for j in range(4):
            s = pl.ds(b + j * _L, _L)
            x = vb[s]
            vb[s] = 1.0 / (1.0 + jnp.exp(-x))
        return c

    lax.fori_loop(0, _C2 // (4 * _L), body, 0)


def _sc_body(idx_hbm, w_hbm, out_hbm, w_sh,
             idx_a, idx_b, vals_a, vals_b, sem_a, sem_b):
    cid = lax.axis_index("c")
    sid = lax.axis_index("s")
    wid = sid * _NC + cid

    tb = sid * _TPT
    stage_bufs = ((vals_a, sem_a), (vals_b, sem_b))
    outcps = []
    for k in range(_TPT // _C2):
        vb, sm = stage_bufs[k % 2]
        if k >= 2:
            outcps[k - 2].wait()
        o = tb + k * _C2
        pltpu.sync_copy(w_hbm.at[pl.ds(o, _C2)], vb)
        outcps.append(pltpu.async_copy(vb, w_sh.at[pl.ds(o, _C2)], sm))
    for cp in outcps[-2:]:
        cp.wait()
    plsc.subcore_barrier()

    per_w = idx_hbm.shape[0] // _NW
    base = wid * per_w
    nch = per_w // _C2
    bufs = ((idx_a, vals_a, sem_a), (idx_b, vals_b, sem_b))

    pltpu.sync_copy(idx_hbm.at[pl.ds(base, _C2)], idx_a)
    pending = (pltpu.async_copy(w_sh.at[idx_a], vals_a, sem_a), base, vals_a)
    for g in range(1, nch):
        ib, vb, sm = bufs[g % 2]
        off = base + g * _C2
        pltpu.sync_copy(idx_hbm.at[pl.ds(off, _C2)], ib)
        src = w_sh if g % 2 == 0 else w_hbm
        cp = pltpu.async_copy(src.at[ib], vb, sm)
        pcp, poff, pvb = pending
        pcp.wait()
        _sigmoid_chunk(pvb)
        pltpu.sync_copy(pvb, out_hbm.at[pl.ds(poff, _C2)])
        pending = (cp, off, vb)
    pcp, poff, pvb = pending
    pcp.wait()
    _sigmoid_chunk(pvb)
    pltpu.sync_copy(pvb, out_hbm.at[pl.ds(poff, _C2)])


def kernel(idx, weight):
    n = idx.shape[0]
    assert n % (_NW * _C2) == 0
    flat_idx = idx.reshape(-1)
    w_pad = jnp.pad(weight, (0, _PAD_V - weight.shape[0]))

    run = pl.kernel(
        _sc_body,
        out_type=jax.ShapeDtypeStruct((n,), jnp.float32),
        mesh=plsc.VectorSubcoreMesh(core_axis_name="c", subcore_axis_name="s"),
        scratch_types=[
            pltpu.VMEM_SHARED((_PAD_V,), jnp.float32),
            pltpu.VMEM((_C2,), jnp.int32),
            pltpu.VMEM((_C2,), jnp.int32),
            pltpu.VMEM((_C2,), jnp.float32),
            pltpu.VMEM((_C2,), jnp.float32),
            pltpu.SemaphoreType.DMA,
            pltpu.SemaphoreType.DMA,
        ],
    )
    out = run(flat_idx, w_pad)
    return out.reshape(idx.shape)


# speedup vs baseline: 1.4587x; 1.4587x over previous
import jax
import jax.numpy as jnp
from jax import lax
from jax.experimental import pallas as pl
from jax.experimental.pallas import tpu as pltpu
from jax.experimental.pallas import tpu_sc as plsc

_NC = 2
_NS = 16
_NW = _NC * _NS
_L = 16

_C2 = 12800
_CT = 3200
_NBIG = 7
_NT = 4
_PAD_V = 1024000
_TPT = _PAD_V // _NS


def _sigmoid_chunk(vb, n):
    def body(i, c):
        b = i * (4 * _L)
        for j in range(4):
            s = pl.ds(b + j * _L, _L)
            x = vb[s]
            vb[s] = 1.0 / (1.0 + jnp.exp(-x))
        return c

    lax.fori_loop(0, n // (4 * _L), body, 0)


def _sc_body(idx_hbm, w_hbm, out_hbm, w_sh,
             idx_a, idx_b, vals_a, vals_b,
             idx_c, idx_d, vals_c, vals_d,
             sem_ga, sem_gb, sem_gc, sem_gd,
             sem_sia, sem_sib, sem_soa, sem_sob,
             sem_ixa, sem_ixb):
    cid = lax.axis_index("c")
    sid = lax.axis_index("s")
    wid = sid * _NC + cid
    per_w = idx_hbm.shape[0] // _NW
    base = wid * per_w

    pre0 = pltpu.async_copy(idx_hbm.at[pl.ds(base, _C2)], idx_a, sem_ixa)
    pre1 = pltpu.async_copy(idx_hbm.at[pl.ds(base + _C2, _C2)], idx_b,
                            sem_ixb)

    tb = sid * _TPT
    stage = ((vals_a, sem_sia, sem_soa), (vals_b, sem_sib, sem_sob))

    def stage_in(k):
        vb, si, _ = stage[k % 2]
        return pltpu.async_copy(w_hbm.at[pl.ds(tb + k * _C2, _C2)], vb, si)

    npiece = _TPT // _C2
    incp = {0: stage_in(0), 1: stage_in(1)}
    last_outs = []
    for k in range(npiece):
        vb, _, so = stage[k % 2]
        incp[k].wait()
        out = pltpu.async_copy(vb, w_sh.at[pl.ds(tb + k * _C2, _C2)], so)
        if k + 2 < npiece:
            out.wait()
            incp[k + 2] = stage_in(k + 2)
        else:
            last_outs.append(out)
    for o in last_outs:
        o.wait()
    plsc.subcore_barrier()

    sched = [(k * _C2, _C2, (idx_a, vals_a, sem_ga) if k % 2 == 0
              else (idx_b, vals_b, sem_gb), k) for k in range(_NBIG)]
    toff = _NBIG * _C2
    sched += [(toff + j * _CT, _CT, (idx_c, vals_c, sem_gc) if j % 2 == 0
               else (idx_d, vals_d, sem_gd), None) for j in range(_NT)]

    pending = None
    for i, (off, sz, (ib, vb, sm), _big) in enumerate(sched):
        if i == 0:
            pre0.wait()
        elif i == 1:
            pre1.wait()
        else:
            pltpu.sync_copy(idx_hbm.at[pl.ds(base + off, sz)], ib)
        cp = pltpu.async_copy(w_sh.at[ib], vb, sm)
        if pending is not None:
            pcp, poff, psz, pvb = pending
            pcp.wait()
            _sigmoid_chunk(pvb, psz)
            pltpu.sync_copy(pvb, out_hbm.at[pl.ds(base + poff, psz)])
        pending = (cp, off, sz, vb)
    pcp, poff, psz, pvb = pending
    pcp.wait()
    _sigmoid_chunk(pvb, psz)
    pltpu.sync_copy(pvb, out_hbm.at[pl.ds(base + poff, psz)])


def kernel(idx, weight):
    n = idx.shape[0]
    assert n % (_NW * (_NBIG * _C2 + _NT * _CT)) == 0 or \
        n == _NW * (_NBIG * _C2 + _NT * _CT)
    flat_idx = idx.reshape(-1)
    w_pad = jnp.pad(weight, (0, _PAD_V - weight.shape[0]))

    run = pl.kernel(
        _sc_body,
        out_type=jax.ShapeDtypeStruct((n,), jnp.float32),
        mesh=plsc.VectorSubcoreMesh(core_axis_name="c", subcore_axis_name="s"),
        scratch_types=[
            pltpu.VMEM_SHARED((_PAD_V,), jnp.float32),
            pltpu.VMEM((_C2,), jnp.int32),
            pltpu.VMEM((_C2,), jnp.int32),
            pltpu.VMEM((_C2,), jnp.float32),
            pltpu.VMEM((_C2,), jnp.float32),
            pltpu.VMEM((_CT,), jnp.int32),
            pltpu.VMEM((_CT,), jnp.int32),
            pltpu.VMEM((_CT,), jnp.float32),
            pltpu.VMEM((_CT,), jnp.float32),
        ] + [pltpu.SemaphoreType.DMA] * 10,
    )
    out = run(flat_idx, w_pad)
    return out.reshape(idx.shape)


# speedup vs baseline: 1.4920x; 1.0228x over previous
import jax
import jax.numpy as jnp
from jax import lax
from jax.experimental import pallas as pl
from jax.experimental.pallas import tpu as pltpu
from jax.experimental.pallas import tpu_sc as plsc

_NC = 2
_NS = 16
_NW = _NC * _NS
_L = 16

_C2 = 12800
_PAD_V = 1024000
_TPT = _PAD_V // _NS


def _sigmoid_chunk(vb):
    def body(i, c):
        b = i * (4 * _L)
        for j in range(4):
            s = pl.ds(b + j * _L, _L)
            x = vb[s]
            vb[s] = 1.0 / (1.0 + jnp.exp(-x))
        return c

    lax.fori_loop(0, _C2 // (4 * _L), body, 0)


def _sc_body(idx_hbm, w_hbm, out_hbm, w_sh,
             idx_a, idx_b, vals_a, vals_b,
             sem_a, sem_b, sem_pa, sem_pb):
    cid = lax.axis_index("c")
    sid = lax.axis_index("s")
    wid = sid * _NC + cid
    per_w = idx_hbm.shape[0] // _NW
    base = wid * per_w

    pre0 = pltpu.async_copy(idx_hbm.at[pl.ds(base, _C2)], idx_a, sem_pa)
    pre1 = pltpu.async_copy(idx_hbm.at[pl.ds(base + _C2, _C2)], idx_b, sem_pb)

    tb = sid * _TPT
    stage_bufs = ((vals_a, sem_a), (vals_b, sem_b))
    outcps = []
    for k in range(_TPT // _C2):
        vb, sm = stage_bufs[k % 2]
        if k >= 2:
            outcps[k - 2].wait()
        o = tb + k * _C2
        pltpu.sync_copy(w_hbm.at[pl.ds(o, _C2)], vb)
        outcps.append(pltpu.async_copy(vb, w_sh.at[pl.ds(o, _C2)], sm))
    for cp in outcps[-2:]:
        cp.wait()
    plsc.subcore_barrier()

    nch = per_w // _C2
    bufs = ((idx_a, vals_a, sem_a), (idx_b, vals_b, sem_b))

    pre0.wait()
    pending = (pltpu.async_copy(w_sh.at[idx_a], vals_a, sem_a), base, vals_a)
    for g in range(1, nch):
        ib, vb, sm = bufs[g % 2]
        off = base + g * _C2
        if g == 1:
            pre1.wait()
        else:
            pltpu.sync_copy(idx_hbm.at[pl.ds(off, _C2)], ib)
        cp = pltpu.async_copy(w_sh.at[ib], vb, sm)
        pcp, poff, pvb = pending
        pcp.wait()
        _sigmoid_chunk(pvb)
        pltpu.sync_copy(pvb, out_hbm.at[pl.ds(poff, _C2)])
        pending = (cp, off, vb)
    pcp, poff, pvb = pending
    pcp.wait()
    _sigmoid_chunk(pvb)
    pltpu.sync_copy(pvb, out_hbm.at[pl.ds(poff, _C2)])


def kernel(idx, weight):
    n = idx.shape[0]
    assert n % (_NW * _C2) == 0
    flat_idx = idx.reshape(-1)
    w_pad = jnp.pad(weight, (0, _PAD_V - weight.shape[0]))

    run = pl.kernel(
        _sc_body,
        out_type=jax.ShapeDtypeStruct((n,), jnp.float32),
        mesh=plsc.VectorSubcoreMesh(core_axis_name="c", subcore_axis_name="s"),
        scratch_types=[
            pltpu.VMEM_SHARED((_PAD_V,), jnp.float32),
            pltpu.VMEM((_C2,), jnp.int32),
            pltpu.VMEM((_C2,), jnp.int32),
            pltpu.VMEM((_C2,), jnp.float32),
            pltpu.VMEM((_C2,), jnp.float32),
            pltpu.SemaphoreType.DMA,
            pltpu.SemaphoreType.DMA,
            pltpu.SemaphoreType.DMA,
            pltpu.SemaphoreType.DMA,
        ],
    )
    out = run(flat_idx, w_pad)
    return out.reshape(idx.shape)


# speedup vs baseline: 1.5071x; 1.0101x over previous
import jax
import jax.numpy as jnp
from jax import lax
from jax.experimental import pallas as pl
from jax.experimental.pallas import tpu as pltpu
from jax.experimental.pallas import tpu_sc as plsc

_NC = 2
_NS = 16
_NW = _NC * _NS
_L = 16

_C2 = 12800
_PAD_V = 1024000
_TPT = _PAD_V // _NS


def _sigmoid_chunk(vb):
    def body(i, c):
        b = i * (4 * _L)
        for j in range(4):
            s = pl.ds(b + j * _L, _L)
            x = vb[s]
            vb[s] = 1.0 / (1.0 + jnp.exp(-x))
        return c

    lax.fori_loop(0, _C2 // (4 * _L), body, 0)


def _sc_body(idx_hbm, w_hbm, out_hbm, w_sh,
             idx_a, idx_b, vals_a, vals_b, sem_a, sem_b):
    cid = lax.axis_index("c")
    sid = lax.axis_index("s")
    wid = sid * _NC + cid

    tb = sid * _TPT
    stage_bufs = ((vals_a, sem_a), (vals_b, sem_b))
    outcps = []
    for k in range(_TPT // _C2):
        vb, sm = stage_bufs[k % 2]
        if k >= 2:
            outcps[k - 2].wait()
        o = tb + k * _C2
        pltpu.sync_copy(w_hbm.at[pl.ds(o, _C2)], vb)
        outcps.append(pltpu.async_copy(vb, w_sh.at[pl.ds(o, _C2)], sm))
    for cp in outcps[-2:]:
        cp.wait()
    plsc.subcore_barrier()

    per_w = idx_hbm.shape[0] // _NW
    base = wid * per_w
    nch = per_w // _C2
    bufs = ((idx_a, vals_a, sem_a), (idx_b, vals_b, sem_b))

    pltpu.sync_copy(idx_hbm.at[pl.ds(base, _C2)], idx_a)
    pending = (pltpu.async_copy(w_sh.at[idx_a], vals_a, sem_a), base, vals_a)
    for g in range(1, nch):
        ib, vb, sm = bufs[g % 2]
        off = base + g * _C2
        pltpu.sync_copy(idx_hbm.at[pl.ds(off, _C2)], ib)
        cp = pltpu.async_copy(w_sh.at[ib], vb, sm)
        pcp, poff, pvb = pending
        pcp.wait()
        _sigmoid_chunk(pvb)
        pltpu.sync_copy(pvb, out_hbm.at[pl.ds(poff, _C2)])
        pending = (cp, off, vb)
    pcp, poff, pvb = pending
    pcp.wait()
    _sigmoid_chunk(pvb)
    pltpu.sync_copy(pvb, out_hbm.at[pl.ds(poff, _C2)])


def kernel(idx, weight):
    n = idx.shape[0]
    assert n % (_NW * _C2) == 0
    flat_idx = idx.reshape(-1)
    w_pad = jnp.pad(weight, (0, _PAD_V - weight.shape[0]))

    run = pl.kernel(
        _sc_body,
        out_type=jax.ShapeDtypeStruct((n,), jnp.float32),
        mesh=plsc.VectorSubcoreMesh(core_axis_name="c", subcore_axis_name="s"),
        scratch_types=[
            pltpu.VMEM_SHARED((_PAD_V,), jnp.float32),
            pltpu.VMEM((_C2,), jnp.int32),
            pltpu.VMEM((_C2,), jnp.int32),
            pltpu.VMEM((_C2,), jnp.float32),
            pltpu.VMEM((_C2,), jnp.float32),
            pltpu.SemaphoreType.DMA,
            pltpu.SemaphoreType.DMA,
        ],
    )
    out = run(flat_idx, w_pad)
    return out.reshape(idx.shape)


# speedup vs baseline: 1.5101x; 1.0020x over previous
import jax
import jax.numpy as jnp
from jax import lax
from jax.experimental import pallas as pl
from jax.experimental.pallas import tpu as pltpu
from jax.experimental.pallas import tpu_sc as plsc

_NC = 2
_NS = 16
_NW = _NC * _NS
_L = 16

_C2 = 12800
_PAD_V = 1024000
_TPT = _PAD_V // _NS


def _sigmoid_chunk(vb):
    def body(i, c):
        b = i * (4 * _L)
        for j in range(4):
            s = pl.ds(b + j * _L, _L)
            x = vb[s]
            vb[s] = 1.0 / (1.0 + jnp.exp(-x))
        return c

    lax.fori_loop(0, _C2 // (4 * _L), body, 0)


def _sc_body(idx_hbm, w_hbm, out_hbm, w_sh,
             idx_a, idx_b, vals_a, vals_b, sem_a, sem_b):
    cid = lax.axis_index("c")
    sid = lax.axis_index("s")
    wid = sid * _NC + cid

    with jax.named_scope("stage_table"):
        tb = sid * _TPT
        stage_bufs = ((vals_a, sem_a), (vals_b, sem_b))
        outcps = []
        for k in range(_TPT // _C2):
            vb, sm = stage_bufs[k % 2]
            if k >= 2:
                outcps[k - 2].wait()
            o = tb + k * _C2
            pltpu.sync_copy(w_hbm.at[pl.ds(o, _C2)], vb)
            outcps.append(pltpu.async_copy(vb, w_sh.at[pl.ds(o, _C2)], sm))
        for cp in outcps[-2:]:
            cp.wait()
        plsc.subcore_barrier()

    per_w = idx_hbm.shape[0] // _NW
    base = wid * per_w
    nch = per_w // _C2
    bufs = ((idx_a, vals_a, sem_a), (idx_b, vals_b, sem_b))

    with jax.named_scope("gather_loop"):
        pltpu.sync_copy(idx_hbm.at[pl.ds(base, _C2)], idx_a)
        pending = (pltpu.async_copy(w_sh.at[idx_a], vals_a, sem_a), base,
                   vals_a)
        for g in range(1, nch):
            ib, vb, sm = bufs[g % 2]
            off = base + g * _C2
            pltpu.sync_copy(idx_hbm.at[pl.ds(off, _C2)], ib)
            cp = pltpu.async_copy(w_sh.at[ib], vb, sm)
            pcp, poff, pvb = pending
            pcp.wait()
            _sigmoid_chunk(pvb)
            pltpu.sync_copy(pvb, out_hbm.at[pl.ds(poff, _C2)])
            pending = (cp, off, vb)
    with jax.named_scope("tail"):
        pcp, poff, pvb = pending
        pcp.wait()
        _sigmoid_chunk(pvb)
        pltpu.sync_copy(pvb, out_hbm.at[pl.ds(poff, _C2)])


def kernel(idx, weight):
    n = idx.shape[0]
    assert n % (_NW * _C2) == 0
    flat_idx = idx.reshape(-1)
    w_pad = jnp.pad(weight, (0, _PAD_V - weight.shape[0]))

    run = pl.kernel(
        _sc_body,
        out_type=jax.ShapeDtypeStruct((n,), jnp.float32),
        mesh=plsc.VectorSubcoreMesh(core_axis_name="c", subcore_axis_name="s"),
        scratch_types=[
            pltpu.VMEM_SHARED((_PAD_V,), jnp.float32),
            pltpu.VMEM((_C2,), jnp.int32),
            pltpu.VMEM((_C2,), jnp.int32),
            pltpu.VMEM((_C2,), jnp.float32),
            pltpu.VMEM((_C2,), jnp.float32),
            pltpu.SemaphoreType.DMA,
            pltpu.SemaphoreType.DMA,
        ],
    )
    out = run(flat_idx, w_pad)
    return out.reshape(idx.shape)


# speedup vs baseline: 1.5309x; 1.0138x over previous
import jax
import jax.numpy as jnp
from jax import lax
from jax.experimental import pallas as pl
from jax.experimental.pallas import tpu as pltpu
from jax.experimental.pallas import tpu_sc as plsc

_NC = 2
_NS = 16
_NW = _NC * _NS
_L = 16

_C2 = 12800


def _sigmoid_chunk(vb):
    def body(i, c):
        b = i * (4 * _L)
        for j in range(4):
            s = pl.ds(b + j * _L, _L)
            x = vb[s]
            vb[s] = 1.0 / (1.0 + jnp.exp(-x))
        return c

    lax.fori_loop(0, _C2 // (4 * _L), body, 0)


def _sc_body(idx_hbm, w_hbm, out_hbm, w_sh,
             idx_a, idx_b, vals_a, vals_b, sem_a, sem_b):
    cid = lax.axis_index("c")
    sid = lax.axis_index("s")
    wid = sid * _NC + cid

    v = w_hbm.shape[0]
    n_pieces = v // _C2
    rem = v - n_pieces * _C2
    n_uniform = n_pieces // _NS
    n_extra = n_pieces - n_uniform * _NS
    stage_bufs = ((vals_a, sem_a), (vals_b, sem_b))
    outcps = []
    for k in range(n_uniform):
        vb, sm = stage_bufs[k % 2]
        if k >= 2:
            outcps[k - 2].wait()
        o = (k * _NS + sid) * _C2
        pltpu.sync_copy(w_hbm.at[pl.ds(o, _C2)], vb)
        outcps.append(pltpu.async_copy(vb, w_sh.at[pl.ds(o, _C2)], sm))
    for cp in outcps[-2:]:
        cp.wait()

    if n_extra > 0:
        @pl.when(sid < n_extra)
        def _():
            o = (n_uniform * _NS + sid) * _C2
            pltpu.sync_copy(w_hbm.at[pl.ds(o, _C2)], vals_a)
            pltpu.async_copy(vals_a, w_sh.at[pl.ds(o, _C2)], sem_a).wait()

    if rem > 0:
        @pl.when(sid == _NS - 1)
        def _():
            o = n_pieces * _C2
            pltpu.sync_copy(w_hbm.at[pl.ds(o, rem)],
                            vals_a.at[pl.ds(0, rem)])
            pltpu.async_copy(vals_a.at[pl.ds(0, rem)],
                             w_sh.at[pl.ds(o, rem)], sem_a).wait()
    plsc.subcore_barrier()

    per_w = idx_hbm.shape[0] // _NW
    base = wid * per_w
    nch = per_w // _C2
    bufs = ((idx_a, vals_a, sem_a), (idx_b, vals_b, sem_b))

    pltpu.sync_copy(idx_hbm.at[pl.ds(base, _C2)], idx_a)
    pending = (pltpu.async_copy(w_sh.at[idx_a], vals_a, sem_a), base, vals_a)
    for g in range(1, nch):
        ib, vb, sm = bufs[g % 2]
        off = base + g * _C2
        pltpu.sync_copy(idx_hbm.at[pl.ds(off, _C2)], ib)
        cp = pltpu.async_copy(w_sh.at[ib], vb, sm)
        pcp, poff, pvb = pending
        pcp.wait()
        _sigmoid_chunk(pvb)
        pltpu.sync_copy(pvb, out_hbm.at[pl.ds(poff, _C2)])
        pending = (cp, off, vb)
    pcp, poff, pvb = pending
    pcp.wait()
    _sigmoid_chunk(pvb)
    pltpu.sync_copy(pvb, out_hbm.at[pl.ds(poff, _C2)])


def kernel(idx, weight):
    n = idx.shape[0]
    assert n % (_NW * _C2) == 0
    assert weight.shape[0] % 8 == 0
    flat_idx = idx.reshape(-1)

    run = pl.kernel(
        _sc_body,
        out_type=jax.ShapeDtypeStruct((n,), jnp.float32),
        mesh=plsc.VectorSubcoreMesh(core_axis_name="c", subcore_axis_name="s"),
        scratch_types=[
            pltpu.VMEM_SHARED((weight.shape[0],), jnp.float32),
            pltpu.VMEM((_C2,), jnp.int32),
            pltpu.VMEM((_C2,), jnp.int32),
            pltpu.VMEM((_C2,), jnp.float32),
            pltpu.VMEM((_C2,), jnp.float32),
            pltpu.SemaphoreType.DMA,
            pltpu.SemaphoreType.DMA,
        ],
    )
    out = run(flat_idx, weight)
    return out.reshape(idx.shape)


# speedup vs baseline: 1.5458x; 1.0097x over previous
import jax
import jax.numpy as jnp
from jax import lax
from jax.experimental import pallas as pl
from jax.experimental.pallas import tpu as pltpu
from jax.experimental.pallas import tpu_sc as plsc

_NC = 2
_NS = 16
_NW = _NC * _NS
_L = 16

_C2 = 12800


def _sigmoid_chunk(vb):
    def body(i, c):
        b = i * (4 * _L)
        for j in range(4):
            s = pl.ds(b + j * _L, _L)
            x = vb[s]
            vb[s] = 1.0 / (1.0 + jnp.exp(-x))
        return c

    lax.fori_loop(0, _C2 // (4 * _L), body, 0)


def _sc_body(idx_hbm, w_hbm, out_hbm, w_sh,
             idx_a, idx_b, vals_a, vals_b, sem_a, sem_b, sem_c, sem_d):
    cid = lax.axis_index("c")
    sid = lax.axis_index("s")
    wid = sid * _NC + cid

    v = w_hbm.shape[0]
    n_pieces = v // _C2
    rem = v - n_pieces * _C2
    n_uniform = n_pieces // _NS
    n_extra = n_pieces - n_uniform * _NS
    stage_bufs = ((vals_a, sem_a, sem_c), (vals_b, sem_b, sem_d))

    def stage_in(k):
        vb, _, si = stage_bufs[k % 2]
        o = (k * _NS + sid) * _C2
        return pltpu.async_copy(w_hbm.at[pl.ds(o, _C2)], vb, si)

    incps = {0: stage_in(0), 1: stage_in(1)}
    outcps = []
    for k in range(n_uniform):
        vb, so, _ = stage_bufs[k % 2]
        incps[k].wait()
        o = (k * _NS + sid) * _C2
        outcps.append(pltpu.async_copy(vb, w_sh.at[pl.ds(o, _C2)], so))
        if k + 2 < n_uniform:
            outcps[k].wait()
            incps[k + 2] = stage_in(k + 2)
    for cp in outcps[-2:]:
        cp.wait()

    if n_extra > 0:
        @pl.when(sid < n_extra)
        def _():
            o = (n_uniform * _NS + sid) * _C2
            pltpu.sync_copy(w_hbm.at[pl.ds(o, _C2)], vals_a)
            pltpu.async_copy(vals_a, w_sh.at[pl.ds(o, _C2)], sem_a).wait()

    if rem > 0:
        @pl.when(sid == _NS - 1)
        def _():
            o = n_pieces * _C2
            pltpu.sync_copy(w_hbm.at[pl.ds(o, rem)],
                            vals_a.at[pl.ds(0, rem)])
            pltpu.async_copy(vals_a.at[pl.ds(0, rem)],
                             w_sh.at[pl.ds(o, rem)], sem_a).wait()
    plsc.subcore_barrier()

    per_w = idx_hbm.shape[0] // _NW
    base = wid * per_w
    nch = per_w // _C2
    bufs = ((idx_a, vals_a, sem_a), (idx_b, vals_b, sem_b))

    pltpu.sync_copy(idx_hbm.at[pl.ds(base, _C2)], idx_a)
    pending = (pltpu.async_copy(w_sh.at[idx_a], vals_a, sem_a), base, vals_a)
    for g in range(1, nch):
        ib, vb, sm = bufs[g % 2]
        off = base + g * _C2
        pltpu.sync_copy(idx_hbm.at[pl.ds(off, _C2)], ib)
        cp = pltpu.async_copy(w_sh.at[ib], vb, sm)
        pcp, poff, pvb = pending
        pcp.wait()
        _sigmoid_chunk(pvb)
        pltpu.sync_copy(pvb, out_hbm.at[pl.ds(poff, _C2)])
        pending = (cp, off, vb)
    pcp, poff, pvb = pending
    pcp.wait()
    _sigmoid_chunk(pvb)
    pltpu.sync_copy(pvb, out_hbm.at[pl.ds(poff, _C2)])


def kernel(idx, weight):
    n = idx.shape[0]
    assert n % (_NW * _C2) == 0
    assert weight.shape[0] % 8 == 0
    flat_idx = idx.reshape(-1)

    run = pl.kernel(
        _sc_body,
        out_type=jax.ShapeDtypeStruct((n,), jnp.float32),
        mesh=plsc.VectorSubcoreMesh(core_axis_name="c", subcore_axis_name="s"),
        scratch_types=[
            pltpu.VMEM_SHARED((weight.shape[0],), jnp.float32),
            pltpu.VMEM((_C2,), jnp.int32),
            pltpu.VMEM((_C2,), jnp.int32),
            pltpu.VMEM((_C2,), jnp.float32),
            pltpu.VMEM((_C2,), jnp.float32),
            pltpu.SemaphoreType.DMA,
            pltpu.SemaphoreType.DMA,
            pltpu.SemaphoreType.DMA,
            pltpu.SemaphoreType.DMA,
        ],
    )
    out = run(flat_idx, weight)
    return out.reshape(idx.shape)


# speedup vs baseline: 1.5677x; 1.0142x over previous
import jax
import jax.numpy as jnp
from jax import lax
from jax.experimental import pallas as pl
from jax.experimental.pallas import tpu as pltpu
from jax.experimental.pallas import tpu_sc as plsc

_NC = 2
_NS = 16
_NW = _NC * _NS
_L = 16

_C2 = 12800


def _sigmoid_chunk(vb):
    def body(i, c):
        b = i * (4 * _L)
        for j in range(4):
            s = pl.ds(b + j * _L, _L)
            x = vb[s]
            vb[s] = 1.0 / (1.0 + jnp.exp(-x))
        return c

    lax.fori_loop(0, _C2 // (4 * _L), body, 0)


def _sc_body(idx_hbm, w_hbm, out_hbm, w_sh,
             idx_a, idx_b, vals_a, vals_b, sem_a, sem_b, sem_c, sem_d):
    cid = lax.axis_index("c")
    sid = lax.axis_index("s")
    wid = sid * _NC + cid

    v = w_hbm.shape[0]
    n_pieces = v // _C2
    rem = v - n_pieces * _C2
    n_uniform = n_pieces // _NS
    n_extra = n_pieces - n_uniform * _NS
    stage_bufs = ((vals_a, sem_a, sem_c), (vals_b, sem_b, sem_d))

    def stage_in(k):
        vb, _, si = stage_bufs[k % 2]
        o = (k * _NS + sid) * _C2
        return pltpu.async_copy(w_hbm.at[pl.ds(o, _C2)], vb, si)

    incps = {0: stage_in(0), 1: stage_in(1)}
    outcps = []
    for k in range(n_uniform):
        vb, so, _ = stage_bufs[k % 2]
        incps[k].wait()
        o = (k * _NS + sid) * _C2
        outcps.append(pltpu.async_copy(vb, w_sh.at[pl.ds(o, _C2)], so))
        if k + 2 < n_uniform:
            outcps[k].wait()
            incps[k + 2] = stage_in(k + 2)
    per_w = idx_hbm.shape[0] // _NW
    base = wid * per_w
    pre0 = pltpu.async_copy(idx_hbm.at[pl.ds(base, _C2)], idx_a, sem_c)

    for cp in outcps[-2:]:
        cp.wait()

    if n_extra > 0:
        @pl.when(sid < n_extra)
        def _():
            o = (n_uniform * _NS + sid) * _C2
            pltpu.sync_copy(w_hbm.at[pl.ds(o, _C2)], vals_a)
            pltpu.async_copy(vals_a, w_sh.at[pl.ds(o, _C2)], sem_a).wait()

    if rem > 0:
        @pl.when(sid == _NS - 1)
        def _():
            o = n_pieces * _C2
            pltpu.sync_copy(w_hbm.at[pl.ds(o, rem)],
                            vals_a.at[pl.ds(0, rem)])
            pltpu.async_copy(vals_a.at[pl.ds(0, rem)],
                             w_sh.at[pl.ds(o, rem)], sem_a).wait()
    plsc.subcore_barrier()

    nch = per_w // _C2
    bufs = ((idx_a, vals_a, sem_a), (idx_b, vals_b, sem_b))

    pre0.wait()
    pending = (pltpu.async_copy(w_sh.at[idx_a], vals_a, sem_a), base, vals_a)
    for g in range(1, nch):
        ib, vb, sm = bufs[g % 2]
        off = base + g * _C2
        pltpu.sync_copy(idx_hbm.at[pl.ds(off, _C2)], ib)
        cp = pltpu.async_copy(w_sh.at[ib], vb, sm)
        pcp, poff, pvb = pending
        pcp.wait()
        _sigmoid_chunk(pvb)
        pltpu.sync_copy(pvb, out_hbm.at[pl.ds(poff, _C2)])
        pending = (cp, off, vb)
    pcp, poff, pvb = pending
    pcp.wait()
    _sigmoid_chunk(pvb)
    pltpu.sync_copy(pvb, out_hbm.at[pl.ds(poff, _C2)])


def kernel(idx, weight):
    n = idx.shape[0]
    assert n % (_NW * _C2) == 0
    assert weight.shape[0] % 8 == 0
    flat_idx = idx.reshape(-1)

    run = pl.kernel(
        _sc_body,
        out_type=jax.ShapeDtypeStruct((n,), jnp.float32),
        mesh=plsc.VectorSubcoreMesh(core_axis_name="c", subcore_axis_name="s"),
        scratch_types=[
            pltpu.VMEM_SHARED((weight.shape[0],), jnp.float32),
            pltpu.VMEM((_C2,), jnp.int32),
            pltpu.VMEM((_C2,), jnp.int32),
            pltpu.VMEM((_C2,), jnp.float32),
            pltpu.VMEM((_C2,), jnp.float32),
            pltpu.SemaphoreType.DMA,
            pltpu.SemaphoreType.DMA,
            pltpu.SemaphoreType.DMA,
            pltpu.SemaphoreType.DMA,
        ],
    )
    out = run(flat_idx, weight)
    return out.reshape(idx.shape)


# speedup vs baseline: 1.5809x; 1.0084x over previous
import jax
import jax.numpy as jnp
from jax import lax
from jax.experimental import pallas as pl
from jax.experimental.pallas import tpu as pltpu
from jax.experimental.pallas import tpu_sc as plsc

_NC = 2
_NS = 16
_NW = _NC * _NS
_L = 16

_C2 = 12800


def _sigmoid_chunk(vb):
    def body(i, c):
        b = i * (4 * _L)
        for j in range(4):
            s = pl.ds(b + j * _L, _L)
            x = vb[s]
            vb[s] = 1.0 / (1.0 + jnp.exp(-x))
        return c

    lax.fori_loop(0, _C2 // (4 * _L), body, 0)


def _sc_body(idx_hbm, w_hbm, out_hbm, w_sh,
             idx_a, idx_b, vals_a, vals_b, vals_e,
             sem_a, sem_b, sem_c, sem_d, sem_e):
    cid = lax.axis_index("c")
    sid = lax.axis_index("s")
    wid = sid * _NC + cid

    v = w_hbm.shape[0]
    n_pieces = v // _C2
    rem = v - n_pieces * _C2
    n_uniform = n_pieces // _NS
    n_extra = n_pieces - n_uniform * _NS
    stage_bufs = ((vals_a, sem_a, sem_c), (vals_b, sem_b, sem_d))

    def stage_in(k):
        vb, _, si = stage_bufs[k % 2]
        o = (k * _NS + sid) * _C2
        return pltpu.async_copy(w_hbm.at[pl.ds(o, _C2)], vb, si)

    incps = {0: stage_in(0), 1: stage_in(1)}

    o_x = (n_uniform * _NS + sid) * _C2
    o_r = n_pieces * _C2
    if n_extra > 0:
        @pl.when(sid < n_extra)
        def _():
            pltpu.async_copy(w_hbm.at[pl.ds(o_x, _C2)], vals_e, sem_e)
    if rem > 0:
        @pl.when(sid == _NS - 1)
        def _():
            pltpu.async_copy(w_hbm.at[pl.ds(o_r, rem)],
                             vals_e.at[pl.ds(0, rem)], sem_e)

    outcps = []
    for k in range(n_uniform):
        vb, so, _ = stage_bufs[k % 2]
        incps[k].wait()
        o = (k * _NS + sid) * _C2
        outcps.append(pltpu.async_copy(vb, w_sh.at[pl.ds(o, _C2)], so))
        if k + 2 < n_uniform:
            outcps[k].wait()
            incps[k + 2] = stage_in(k + 2)
    per_w = idx_hbm.shape[0] // _NW
    base = wid * per_w
    pre0 = pltpu.async_copy(idx_hbm.at[pl.ds(base, _C2)], idx_a, sem_c)

    for cp in outcps[-2:]:
        cp.wait()

    if n_extra > 0:
        @pl.when(sid < n_extra)
        def _():
            pltpu.make_async_copy(w_hbm.at[pl.ds(o_x, _C2)], vals_e,
                                  sem_e).wait()
            pltpu.async_copy(vals_e, w_sh.at[pl.ds(o_x, _C2)], sem_e).wait()

    if rem > 0:
        @pl.when(sid == _NS - 1)
        def _():
            pltpu.make_async_copy(w_hbm.at[pl.ds(o_r, rem)],
                                  vals_e.at[pl.ds(0, rem)], sem_e).wait()
            pltpu.async_copy(vals_e.at[pl.ds(0, rem)],
                             w_sh.at[pl.ds(o_r, rem)], sem_e).wait()
    plsc.subcore_barrier()

    nch = per_w // _C2
    bufs = ((idx_a, vals_a, sem_a), (idx_b, vals_b, sem_b))

    pre0.wait()
    pending = (pltpu.async_copy(w_sh.at[idx_a], vals_a, sem_a), base, vals_a)
    for g in range(1, nch):
        ib, vb, sm = bufs[g % 2]
        off = base + g * _C2
        pltpu.sync_copy(idx_hbm.at[pl.ds(off, _C2)], ib)
        cp = pltpu.async_copy(w_sh.at[ib], vb, sm)
        pcp, poff, pvb = pending
        pcp.wait()
        _sigmoid_chunk(pvb)
        pltpu.sync_copy(pvb, out_hbm.at[pl.ds(poff, _C2)])
        pending = (cp, off, vb)
    pcp, poff, pvb = pending
    pcp.wait()
    _sigmoid_chunk(pvb)
    pltpu.sync_copy(pvb, out_hbm.at[pl.ds(poff, _C2)])


def kernel(idx, weight):
    n = idx.shape[0]
    assert n % (_NW * _C2) == 0
    assert weight.shape[0] % 8 == 0
    flat_idx = idx.reshape(-1)

    run = pl.kernel(
        _sc_body,
        out_type=jax.ShapeDtypeStruct((n,), jnp.float32),
        mesh=plsc.VectorSubcoreMesh(core_axis_name="c", subcore_axis_name="s"),
        scratch_types=[
            pltpu.VMEM_SHARED((weight.shape[0],), jnp.float32),
            pltpu.VMEM((_C2,), jnp.int32),
            pltpu.VMEM((_C2,), jnp.int32),
            pltpu.VMEM((_C2,), jnp.float32),
            pltpu.VMEM((_C2,), jnp.float32),
            pltpu.VMEM((_C2,), jnp.float32),
            pltpu.SemaphoreType.DMA,
            pltpu.SemaphoreType.DMA,
            pltpu.SemaphoreType.DMA,
            pltpu.SemaphoreType.DMA,
            pltpu.SemaphoreType.DMA,
        ],
    )
    out = run(flat_idx, weight)
    return out.reshape(idx.shape)
